# baseline (speedup 1.0000x reference)
import functools

import jax
import jax.numpy as jnp
from jax import lax
from jax.experimental import pallas as pl
from jax.experimental.pallas import tpu as pltpu
from jax.experimental.pallas import tpu_sc as plsc

B, M, D, HOPS = 32, 4096, 128, 3
BSZ = 4
SC_CH = 512


def _sc_stream_body(m_ref, out_ref, b0, b1, s0, s1):
    c = lax.axis_index("c")
    s = lax.axis_index("s")
    wid = s * 2 + c
    bufs = (b0, b1)
    sems = (s0, s1)
    descs = []
    for i in range(M // SC_CH):
        if i >= 2:
            descs[i - 2].wait()
        descs.append(
            pltpu.async_copy(m_ref.at[wid, pl.ds(i * SC_CH, SC_CH)],
                             bufs[i % 2], sems[i % 2]))
    descs[-2].wait()
    descs[-1].wait()
    pltpu.sync_copy(bufs[1].at[0, pl.ds(0, 16)], out_ref.at[wid])


def _sc_stream(m):
    mesh = plsc.VectorSubcoreMesh(core_axis_name="c", subcore_axis_name="s")
    return pl.kernel(
        _sc_stream_body,
        out_type=jax.ShapeDtypeStruct((B, 16), jnp.float32),
        mesh=mesh,
        scratch_types=[
            pltpu.VMEM((SC_CH, D), jnp.float32),
            pltpu.VMEM((SC_CH, D), jnp.float32),
            pltpu.SemaphoreType.DMA,
            pltpu.SemaphoreType.DMA,
        ],
    )(m)


def _logits(a, u, g):
    t = jax.lax.dot_general(u, a, (((1,), (1,)), ((), ())),
                            preferred_element_type=jnp.float32)
    return t * g


def _body(q_ref, g_ref, m0_ref, m1_ref, m2_ref, soft_ref, logits_ref):
    for b in range(BSZ):
        u = q_ref[0, b][None, :]
        g = g_ref[0, b][None, :]
        a0 = m0_ref[b]
        a1 = m1_ref[b]
        a2 = m2_ref[b]

        def hop(a_logits, a_next, u, g):
            l = _logits(a_logits, u, g)
            e = jnp.exp(l - jnp.max(l, axis=1, keepdims=True))
            eg = e * g
            o = jax.lax.dot_general(eg, a_next, (((1,), (0,)), ((), ())),
                                    preferred_element_type=jnp.float32)
            return u + o / jnp.sum(e, axis=1, keepdims=True)

        u = hop(a0, a1, u, g)
        u = hop(a1, a2, u, g)
        l = _logits(a2, u, g)
        e = jnp.exp(l - jnp.max(l, axis=1, keepdims=True))
        p = e / jnp.sum(e, axis=1, keepdims=True)
        soft_ref[0, b] = p[0]
        logits_ref[0, b] = l[0]


@jax.jit
def kernel(query_vector, global_pointer, m0, m1, m2, m3):
    sc_out = _sc_stream(m3)
    out = pl.pallas_call(
        _body,
        grid=(B // BSZ,),
        in_specs=[
            pl.BlockSpec((1, BSZ, D), lambda i: (i, 0, 0)),
            pl.BlockSpec((1, BSZ, M), lambda i: (i, 0, 0)),
            pl.BlockSpec((BSZ, M, D), lambda i: (i, 0, 0)),
            pl.BlockSpec((BSZ, M, D), lambda i: (i, 0, 0)),
            pl.BlockSpec((BSZ, M, D), lambda i: (i, 0, 0)),
        ],
        out_specs=[
            pl.BlockSpec((1, BSZ, M), lambda i: (i, 0, 0)),
            pl.BlockSpec((1, BSZ, M), lambda i: (i, 0, 0)),
        ],
        out_shape=[
            jax.ShapeDtypeStruct((B // BSZ, BSZ, M), jnp.float32),
            jax.ShapeDtypeStruct((B // BSZ, BSZ, M), jnp.float32),
        ],
    )(query_vector.reshape(B // BSZ, BSZ, D),
      global_pointer.reshape(B // BSZ, BSZ, M), m0, m1, m2)
    tie = sc_out[0, 0] * 1e-30
    return (out[0].reshape(B, M), out[1].reshape(B, M) + tie)


# speedup vs baseline: 1.1119x; 1.1119x over previous
import functools

import jax
import jax.numpy as jnp
from jax import lax
from jax.experimental import pallas as pl
from jax.experimental.pallas import tpu as pltpu
from jax.experimental.pallas import tpu_sc as plsc

B, M, D, HOPS = 32, 4096, 128, 3
BSZ = 4
SC_K = 8
NT = 16
RPT = M // NT
DG = D // 16
XW = 16 + D


_GDN = lax.GatherDimensionNumbers(offset_dims=(), collapsed_slice_dims=(0,),
                                  start_index_map=(0,))


def _shuffle(x, idx):
    return lax.gather(x, idx[:, None], _GDN, slice_sizes=(1,),
                      mode=lax.GatherScatterMode.PROMISE_IN_BOUNDS)


def _lane_sum(x):
    lane = lax.iota(jnp.int32, 16)
    for sh in (1, 2, 4, 8):
        x = x + _shuffle(x, lane ^ sh)
    return x


def _lane_max(x):
    lane = lax.iota(jnp.int32, 16)
    for sh in (1, 2, 4, 8):
        x = jnp.maximum(x, _shuffle(x, lane ^ sh))
    return x


def _sc_body(q_hbm, g_hbm, m0_hbm, m1_hbm, m2_hbm,
             soft_hbm, logits_hbm,
             m0s, m1s, m2s, gsl, lsl, esl, wsl, uv, xv, xo, xb, xob,
             sh_max, sh_os,
             s0, s1, s2, sg, sq):
    c = lax.axis_index("c")
    t = lax.axis_index("s")
    r0 = t * RPT

    def load_u():
        return tuple(uv[pl.ds(16 * j, 16)] for j in range(DG))

    def logits_pass(aref, u):
        lane = lax.iota(jnp.int32, 16)

        def grp(gg, _):
            base = gg * 16
            cur = jnp.full((16,), 0.0, dtype=jnp.float32)
            for k in range(16):
                rr = base + k
                acc = aref[rr, pl.ds(0, 16)] * u[0]
                for j in range(1, DG):
                    acc = acc + aref[rr, pl.ds(16 * j, 16)] * u[j]
                cur = jnp.where(lane == k, _lane_sum(acc), cur)
            lsl[pl.ds(base, 16)] = cur
            return 0
        lax.fori_loop(0, RPT // 16, grp, 0)

    def scale_and_max():
        mx = jnp.full((16,), -jnp.inf, dtype=jnp.float32)
        for j in range(RPT // 16):
            v = lsl[pl.ds(16 * j, 16)] * gsl[pl.ds(16 * j, 16)]
            lsl[pl.ds(16 * j, 16)] = v
            mx = jnp.maximum(mx, v)
        return _lane_max(mx)

    def global_max(mloc):
        xv[...] = mloc
        pltpu.sync_copy(xv, sh_max.at[pl.ds(t * 16, 16)])
        plsc.subcore_barrier()
        pltpu.sync_copy(sh_max, xb)
        gm = xb[pl.ds(0, 16)]
        for tt in range(1, NT):
            gm = jnp.maximum(gm, xb[pl.ds(tt * 16, 16)])
        plsc.subcore_barrier()
        return gm

    def exp_and_localsum(gmax):
        sm = jnp.full((16,), 0.0, dtype=jnp.float32)
        for j in range(RPT // 16):
            e = jnp.exp(lsl[pl.ds(16 * j, 16)] - gmax)
            esl[pl.ds(16 * j, 16)] = e
            sm = sm + e
        return _lane_sum(sm)

    def ok_pass(aref):
        for j in range(RPT // 16):
            wsl[pl.ds(16 * j, 16)] = (esl[pl.ds(16 * j, 16)] *
                                      gsl[pl.ds(16 * j, 16)])

        def grp(gg, accs):
            base = gg * 16
            wv = wsl[pl.ds(base, 16)]
            for k in range(16):
                rr = base + k
                w = wv[k]
                accs = tuple(accs[j] + aref[rr, pl.ds(16 * j, 16)] * w
                             for j in range(DG))
            return accs
        zero = jnp.full((16,), 0.0, dtype=jnp.float32)
        return lax.fori_loop(0, RPT // 16, grp, (zero,) * DG)

    def exchange_sum_o(sloc, accs):
        xo[pl.ds(0, 16)] = sloc
        for j in range(DG):
            xo[pl.ds(16 + 16 * j, 16)] = accs[j]
        pltpu.sync_copy(xo, sh_os.at[pl.ds(t * XW, XW)])
        plsc.subcore_barrier()
        pltpu.sync_copy(sh_os, xob)
        svec = xob[pl.ds(0, 16)]
        for tt in range(1, NT):
            svec = svec + xob[pl.ds(tt * XW, 16)]
        ovec = []
        for j in range(DG):
            o = xob[pl.ds(16 + 16 * j, 16)]
            for tt in range(1, NT):
                o = o + xob[pl.ds(tt * XW + 16 + 16 * j, 16)]
            ovec.append(o)
        plsc.subcore_barrier()
        return svec, ovec

    def batch_body(i, _):
        b = (B - SC_K) + c + 2 * i
        rb = c + 2 * i
        cp0 = pltpu.async_copy(m0_hbm.at[b, pl.ds(r0, RPT)], m0s, s0)
        cp1 = pltpu.async_copy(m1_hbm.at[b, pl.ds(r0, RPT)], m1s, s1)
        cp2 = pltpu.async_copy(m2_hbm.at[b, pl.ds(r0, RPT)], m2s, s2)
        cpg = pltpu.async_copy(g_hbm.at[b, pl.ds(r0, RPT)], gsl, sg)
        cpq = pltpu.async_copy(q_hbm.at[b], uv, sq)
        cpq.wait()
        cpg.wait()
        cp0.wait()

        for hop, (aref_l, aref_n, cp_n) in enumerate(
                ((m0s, m1s, cp1), (m1s, m2s, cp2), (m2s, None, None))):
            logits_pass(aref_l, load_u())
            mloc = scale_and_max()
            gmax = global_max(mloc)
            sloc = exp_and_localsum(gmax)
            if hop < HOPS - 1:
                cp_n.wait()
                accs = ok_pass(aref_n)
                svec, ovec = exchange_sum_o(sloc, accs)
                for j in range(DG):
                    uvj = uv[pl.ds(16 * j, 16)]
                    uv[pl.ds(16 * j, 16)] = uvj + ovec[j] / svec
            else:
                svec, _ = exchange_sum_o(sloc, (jnp.full((16,), 0.0,
                                                dtype=jnp.float32),) * DG)
                for j in range(RPT // 16):
                    esl[pl.ds(16 * j, 16)] = esl[pl.ds(16 * j, 16)] / svec
                pltpu.sync_copy(esl, soft_hbm.at[rb, pl.ds(r0, RPT)])
                pltpu.sync_copy(lsl, logits_hbm.at[rb, pl.ds(r0, RPT)])
        return 0

    lax.fori_loop(0, SC_K // 2, batch_body, 0)


def _sc_call(query_vector, global_pointer, m0, m1, m2):
    mesh = plsc.VectorSubcoreMesh(core_axis_name="c", subcore_axis_name="s",
                                  num_cores=2, num_subcores=NT)
    f32 = jnp.float32
    return pl.kernel(
        _sc_body,
        out_type=[jax.ShapeDtypeStruct((SC_K, M), f32),
                  jax.ShapeDtypeStruct((SC_K, M), f32)],
        mesh=mesh,
        scratch_types=[
            pltpu.VMEM((RPT, D), f32),
            pltpu.VMEM((RPT, D), f32),
            pltpu.VMEM((RPT, D), f32),
            pltpu.VMEM((RPT,), f32),
            pltpu.VMEM((RPT,), f32),
            pltpu.VMEM((RPT,), f32),
            pltpu.VMEM((RPT,), f32),
            pltpu.VMEM((D,), f32),
            pltpu.VMEM((16,), f32),
            pltpu.VMEM((XW,), f32),
            pltpu.VMEM((NT * 16,), f32),
            pltpu.VMEM((NT * XW,), f32),
            pltpu.VMEM_SHARED((NT * 16,), f32),
            pltpu.VMEM_SHARED((NT * XW,), f32),
            pltpu.SemaphoreType.DMA,
            pltpu.SemaphoreType.DMA,
            pltpu.SemaphoreType.DMA,
            pltpu.SemaphoreType.DMA,
            pltpu.SemaphoreType.DMA,
        ],
    )(query_vector, global_pointer, m0, m1, m2)


def _logits(a, u, g):
    t = jax.lax.dot_general(u, a, (((1,), (1,)), ((), ())),
                            preferred_element_type=jnp.float32)
    return t * g


def _tc_body(q_ref, g_ref, m0_ref, m1_ref, m2_ref, soft_ref, logits_ref):
    for b in range(BSZ):
        u = q_ref[0, b][None, :]
        g = g_ref[0, b][None, :]
        a0 = m0_ref[b]
        a1 = m1_ref[b]
        a2 = m2_ref[b]

        def hop(a_logits, a_next, u, g):
            l = _logits(a_logits, u, g)
            e = jnp.exp(l - jnp.max(l, axis=1, keepdims=True))
            eg = e * g
            o = jax.lax.dot_general(eg, a_next, (((1,), (0,)), ((), ())),
                                    preferred_element_type=jnp.float32)
            return u + o / jnp.sum(e, axis=1, keepdims=True)

        u = hop(a0, a1, u, g)
        u = hop(a1, a2, u, g)
        l = _logits(a2, u, g)
        e = jnp.exp(l - jnp.max(l, axis=1, keepdims=True))
        p = e / jnp.sum(e, axis=1, keepdims=True)
        soft_ref[0, b] = p[0]
        logits_ref[0, b] = l[0]


def _tc_call(query_vector, global_pointer, m0, m1, m2):
    nb = B - SC_K
    out = pl.pallas_call(
        _tc_body,
        grid=(nb // BSZ,),
        in_specs=[
            pl.BlockSpec((1, BSZ, D), lambda i: (i, 0, 0)),
            pl.BlockSpec((1, BSZ, M), lambda i: (i, 0, 0)),
            pl.BlockSpec((BSZ, M, D), lambda i: (i, 0, 0)),
            pl.BlockSpec((BSZ, M, D), lambda i: (i, 0, 0)),
            pl.BlockSpec((BSZ, M, D), lambda i: (i, 0, 0)),
        ],
        out_specs=[
            pl.BlockSpec((1, BSZ, M), lambda i: (i, 0, 0)),
            pl.BlockSpec((1, BSZ, M), lambda i: (i, 0, 0)),
        ],
        out_shape=[
            jax.ShapeDtypeStruct((nb // BSZ, BSZ, M), jnp.float32),
            jax.ShapeDtypeStruct((nb // BSZ, BSZ, M), jnp.float32),
        ],
    )(query_vector[:nb].reshape(nb // BSZ, BSZ, D),
      global_pointer[:nb].reshape(nb // BSZ, BSZ, M), m0, m1, m2)
    return out[0].reshape(nb, M), out[1].reshape(nb, M)


@jax.jit
def kernel(query_vector, global_pointer, m0, m1, m2, m3):
    del m3
    sc_soft, sc_logits = _sc_call(query_vector, global_pointer, m0, m1, m2)
    tc_soft, tc_logits = _tc_call(query_vector, global_pointer, m0, m1, m2)
    return (jnp.concatenate([tc_soft, sc_soft], axis=0),
            jnp.concatenate([tc_logits, sc_logits], axis=0))


# speedup vs baseline: 1.1195x; 1.0069x over previous
import functools

import jax
import jax.numpy as jnp
from jax import lax
from jax.experimental import pallas as pl
from jax.experimental.pallas import tpu as pltpu
from jax.experimental.pallas import tpu_sc as plsc

B, M, D, HOPS = 32, 4096, 128, 3
BSZ = 4
SC_K = 8
NT = 16
RPT = M // NT
DG = D // 16
XW = 16 + D


_GDN = lax.GatherDimensionNumbers(offset_dims=(), collapsed_slice_dims=(0,),
                                  start_index_map=(0,))


def _shuffle(x, idx):
    return lax.gather(x, idx[:, None], _GDN, slice_sizes=(1,),
                      mode=lax.GatherScatterMode.PROMISE_IN_BOUNDS)


def _lane_sum(x):
    lane = lax.iota(jnp.int32, 16)
    for sh in (1, 2, 4, 8):
        x = x + _shuffle(x, lane ^ sh)
    return x


def _lane_max(x):
    lane = lax.iota(jnp.int32, 16)
    for sh in (1, 2, 4, 8):
        x = jnp.maximum(x, _shuffle(x, lane ^ sh))
    return x


def _sc_body(q_hbm, g_hbm, m0_hbm, m1_hbm, m2_hbm,
             soft_hbm, logits_hbm,
             m0s, m1s, m2s, gsl, lsl, esl, wsl, uv, xv, xo, xb, xob,
             sh_max, sh_os,
             s0, s1, s2, sg, sq):
    c = lax.axis_index("c")
    t = lax.axis_index("s")
    r0 = t * RPT

    def load_u():
        return tuple(uv[pl.ds(16 * j, 16)] for j in range(DG))

    def logits_pass(aref, u):
        lane = lax.iota(jnp.int32, 16)

        def grp(gg, _):
            base = gg * 16
            vecs = []
            for k in range(16):
                rr = base + k
                prods = [aref[rr, pl.ds(16 * j, 16)] * u[j]
                         for j in range(DG)]
                while len(prods) > 1:
                    prods = [prods[i] + prods[i + 1]
                             for i in range(0, len(prods), 2)]
                vecs.append(prods[0])
            for k in (1, 2, 4, 8):
                m = (lane & k) != 0
                nxt = []
                for i in range(0, len(vecs), 2):
                    x, y = vecs[i], vecs[i + 1]
                    xp = x + _shuffle(x, lane ^ k)
                    yp = y + _shuffle(y, lane ^ k)
                    nxt.append(jnp.where(m, yp, xp))
                vecs = nxt
            lsl[pl.ds(base, 16)] = vecs[0]
            return 0
        lax.fori_loop(0, RPT // 16, grp, 0)

    def scale_and_max():
        mx = jnp.full((16,), -jnp.inf, dtype=jnp.float32)
        for j in range(RPT // 16):
            v = lsl[pl.ds(16 * j, 16)] * gsl[pl.ds(16 * j, 16)]
            lsl[pl.ds(16 * j, 16)] = v
            mx = jnp.maximum(mx, v)
        return _lane_max(mx)

    def global_max(mloc):
        xv[...] = mloc
        pltpu.sync_copy(xv, sh_max.at[pl.ds(t * 16, 16)])
        plsc.subcore_barrier()
        pltpu.sync_copy(sh_max, xb)
        gm = xb[pl.ds(0, 16)]
        for tt in range(1, NT):
            gm = jnp.maximum(gm, xb[pl.ds(tt * 16, 16)])
        plsc.subcore_barrier()
        return gm

    def exp_and_localsum(gmax):
        sm = jnp.full((16,), 0.0, dtype=jnp.float32)
        for j in range(RPT // 16):
            e = jnp.exp(lsl[pl.ds(16 * j, 16)] - gmax)
            esl[pl.ds(16 * j, 16)] = e
            sm = sm + e
        return _lane_sum(sm)

    def ok_pass(aref):
        for j in range(RPT // 16):
            wsl[pl.ds(16 * j, 16)] = (esl[pl.ds(16 * j, 16)] *
                                      gsl[pl.ds(16 * j, 16)])

        def grp(gg, carry):
            a_acc, b_acc = carry
            base = gg * 16
            wv = wsl[pl.ds(base, 16)]
            for k in range(16):
                rr = base + k
                w = wv[k]
                if k % 2 == 0:
                    a_acc = tuple(a_acc[j] + aref[rr, pl.ds(16 * j, 16)] * w
                                  for j in range(DG))
                else:
                    b_acc = tuple(b_acc[j] + aref[rr, pl.ds(16 * j, 16)] * w
                                  for j in range(DG))
            return a_acc, b_acc
        zero = jnp.full((16,), 0.0, dtype=jnp.float32)
        a_acc, b_acc = lax.fori_loop(0, RPT // 16, grp,
                                     ((zero,) * DG, (zero,) * DG))
        return tuple(a_acc[j] + b_acc[j] for j in range(DG))

    def exchange_sum_o(sloc, accs):
        xo[pl.ds(0, 16)] = sloc
        for j in range(DG):
            xo[pl.ds(16 + 16 * j, 16)] = accs[j]
        pltpu.sync_copy(xo, sh_os.at[pl.ds(t * XW, XW)])
        plsc.subcore_barrier()
        pltpu.sync_copy(sh_os, xob)
        svec = xob[pl.ds(0, 16)]
        for tt in range(1, NT):
            svec = svec + xob[pl.ds(tt * XW, 16)]
        ovec = []
        for j in range(DG):
            o = xob[pl.ds(16 + 16 * j, 16)]
            for tt in range(1, NT):
                o = o + xob[pl.ds(tt * XW + 16 + 16 * j, 16)]
            ovec.append(o)
        plsc.subcore_barrier()
        return svec, ovec

    def batch_body(i, _):
        b = (B - SC_K) + c + 2 * i
        rb = c + 2 * i
        cp0 = pltpu.async_copy(m0_hbm.at[b, pl.ds(r0, RPT)], m0s, s0)
        cp1 = pltpu.async_copy(m1_hbm.at[b, pl.ds(r0, RPT)], m1s, s1)
        cp2 = pltpu.async_copy(m2_hbm.at[b, pl.ds(r0, RPT)], m2s, s2)
        cpg = pltpu.async_copy(g_hbm.at[b, pl.ds(r0, RPT)], gsl, sg)
        cpq = pltpu.async_copy(q_hbm.at[b], uv, sq)
        cpq.wait()
        cpg.wait()
        cp0.wait()

        for hop, (aref_l, aref_n, cp_n) in enumerate(
                ((m0s, m1s, cp1), (m1s, m2s, cp2), (m2s, None, None))):
            logits_pass(aref_l, load_u())
            mloc = scale_and_max()
            gmax = global_max(mloc)
            sloc = exp_and_localsum(gmax)
            if hop < HOPS - 1:
                cp_n.wait()
                accs = ok_pass(aref_n)
                svec, ovec = exchange_sum_o(sloc, accs)
                rinv = 1.0 / svec
                for j in range(DG):
                    uvj = uv[pl.ds(16 * j, 16)]
                    uv[pl.ds(16 * j, 16)] = uvj + ovec[j] * rinv
            else:
                svec, _ = exchange_sum_o(sloc, (jnp.full((16,), 0.0,
                                                dtype=jnp.float32),) * DG)
                rinv = 1.0 / svec
                for j in range(RPT // 16):
                    esl[pl.ds(16 * j, 16)] = esl[pl.ds(16 * j, 16)] * rinv
                pltpu.sync_copy(esl, soft_hbm.at[rb, pl.ds(r0, RPT)])
                pltpu.sync_copy(lsl, logits_hbm.at[rb, pl.ds(r0, RPT)])
        return 0

    lax.fori_loop(0, SC_K // 2, batch_body, 0)


def _sc_call(query_vector, global_pointer, m0, m1, m2):
    mesh = plsc.VectorSubcoreMesh(core_axis_name="c", subcore_axis_name="s",
                                  num_cores=2, num_subcores=NT)
    f32 = jnp.float32
    return pl.kernel(
        _sc_body,
        out_type=[jax.ShapeDtypeStruct((SC_K, M), f32),
                  jax.ShapeDtypeStruct((SC_K, M), f32)],
        mesh=mesh,
        scratch_types=[
            pltpu.VMEM((RPT, D), f32),
            pltpu.VMEM((RPT, D), f32),
            pltpu.VMEM((RPT, D), f32),
            pltpu.VMEM((RPT,), f32),
            pltpu.VMEM((RPT,), f32),
            pltpu.VMEM((RPT,), f32),
            pltpu.VMEM((RPT,), f32),
            pltpu.VMEM((D,), f32),
            pltpu.VMEM((16,), f32),
            pltpu.VMEM((XW,), f32),
            pltpu.VMEM((NT * 16,), f32),
            pltpu.VMEM((NT * XW,), f32),
            pltpu.VMEM_SHARED((NT * 16,), f32),
            pltpu.VMEM_SHARED((NT * XW,), f32),
            pltpu.SemaphoreType.DMA,
            pltpu.SemaphoreType.DMA,
            pltpu.SemaphoreType.DMA,
            pltpu.SemaphoreType.DMA,
            pltpu.SemaphoreType.DMA,
        ],
    )(query_vector, global_pointer, m0, m1, m2)


def _logits(a, u, g):
    t = jax.lax.dot_general(u, a, (((1,), (1,)), ((), ())),
                            preferred_element_type=jnp.float32)
    return t * g


def _tc_body(q_ref, g_ref, m0_ref, m1_ref, m2_ref, soft_ref, logits_ref):
    for b in range(BSZ):
        u = q_ref[0, b][None, :]
        g = g_ref[0, b][None, :]
        a0 = m0_ref[b]
        a1 = m1_ref[b]
        a2 = m2_ref[b]

        def hop(a_logits, a_next, u, g):
            l = _logits(a_logits, u, g)
            e = jnp.exp(l - jnp.max(l, axis=1, keepdims=True))
            eg = e * g
            o = jax.lax.dot_general(eg, a_next, (((1,), (0,)), ((), ())),
                                    preferred_element_type=jnp.float32)
            return u + o / jnp.sum(e, axis=1, keepdims=True)

        u = hop(a0, a1, u, g)
        u = hop(a1, a2, u, g)
        l = _logits(a2, u, g)
        e = jnp.exp(l - jnp.max(l, axis=1, keepdims=True))
        p = e / jnp.sum(e, axis=1, keepdims=True)
        soft_ref[0, b] = p[0]
        logits_ref[0, b] = l[0]


def _tc_call(query_vector, global_pointer, m0, m1, m2):
    nb = B - SC_K
    out = pl.pallas_call(
        _tc_body,
        grid=(nb // BSZ,),
        in_specs=[
            pl.BlockSpec((1, BSZ, D), lambda i: (i, 0, 0)),
            pl.BlockSpec((1, BSZ, M), lambda i: (i, 0, 0)),
            pl.BlockSpec((BSZ, M, D), lambda i: (i, 0, 0)),
            pl.BlockSpec((BSZ, M, D), lambda i: (i, 0, 0)),
            pl.BlockSpec((BSZ, M, D), lambda i: (i, 0, 0)),
        ],
        out_specs=[
            pl.BlockSpec((1, BSZ, M), lambda i: (i, 0, 0)),
            pl.BlockSpec((1, BSZ, M), lambda i: (i, 0, 0)),
        ],
        out_shape=[
            jax.ShapeDtypeStruct((nb // BSZ, BSZ, M), jnp.float32),
            jax.ShapeDtypeStruct((nb // BSZ, BSZ, M), jnp.float32),
        ],
    )(query_vector[:nb].reshape(nb // BSZ, BSZ, D),
      global_pointer[:nb].reshape(nb // BSZ, BSZ, M), m0, m1, m2)
    return out[0].reshape(nb, M), out[1].reshape(nb, M)


@jax.jit
def kernel(query_vector, global_pointer, m0, m1, m2, m3):
    del m3
    sc_soft, sc_logits = _sc_call(query_vector, global_pointer, m0, m1, m2)
    tc_soft, tc_logits = _tc_call(query_vector, global_pointer, m0, m1, m2)
    return (jnp.concatenate([tc_soft, sc_soft], axis=0),
            jnp.concatenate([tc_logits, sc_logits], axis=0))
